# Initial kernel scaffold; baseline (speedup 1.0000x reference)
import jax
import jax.numpy as jnp
from jax.experimental import pallas as pl


def kernel(x, edge_index, W1, b1, W2, b2):
    raise NotImplementedError("write your pallas kernel here")


# speedup vs baseline: 2.3749x; 2.3749x over previous
import functools

import jax
import jax.numpy as jnp
from jax import lax
from jax.experimental import pallas as pl
from jax.experimental.pallas import tpu as pltpu
from jax.experimental.pallas import tpu_sc as plsc

N = 10000
E = 320000
D = 128
NC = 2
NS = 16
NW = NC * NS
RANGE = 320
NP = NW * RANGE
TRASH = RANGE
CAP = 16384
G = 128
EC = 4000

_mesh = plsc.VectorSubcoreMesh(core_axis_name="c", subcore_axis_name="s")


def _wid():
    return lax.axis_index("s") * NC + lax.axis_index("c")


def _bin_body(eidx, srcb_hbm, dstb_hbm, cnt_hbm, srcbin, dstbin, sstage,
              dstage, cntv):
    wid = _wid()
    lo = wid * RANGE
    hi = lo + RANGE
    iota = lax.iota(jnp.int32, 16)

    def outer(k, off):
        base = k * EC
        pltpu.sync_copy(eidx.at[pl.ds(base, EC)], sstage)
        pltpu.sync_copy(eidx.at[pl.ds(E + base, EC)], dstage)

        def inner(g, off):
            sv = sstage[pl.ds(g * 16, 16)]
            dv = dstage[pl.ds(g * 16, 16)]
            m = (dv >= lo) & (dv < hi)
            cs = jnp.cumsum(m.astype(jnp.int32))
            pos = jnp.minimum(off + cs - 1, CAP - 1)
            plsc.store_scatter(srcbin, [pos], sv, mask=m)
            plsc.store_scatter(dstbin, [pos], dv - lo, mask=m)
            return off + plsc.all_reduce_population_count(m)

        return lax.fori_loop(0, EC // 16, inner, off)

    off = lax.fori_loop(0, E // EC, outer, jnp.zeros((16,), jnp.int32))

    pcnt = ((off + (G - 1)) // G) * G
    for k in range(G // 16):
        idx = off + k * 16 + iota
        m = idx < pcnt
        cidx = jnp.minimum(idx, CAP - 1)
        plsc.store_scatter(srcbin, [cidx], jnp.zeros((16,), jnp.int32),
                           mask=m)
        plsc.store_scatter(dstbin, [cidx],
                           jnp.full((16,), TRASH, jnp.int32), mask=m)
    cntv[...] = pcnt
    pltpu.sync_copy(cntv, cnt_hbm.at[wid])
    pltpu.sync_copy(srcbin, srcb_hbm.at[wid])
    pltpu.sync_copy(dstbin, dstb_hbm.at[wid])


_bin_call = pl.kernel(
    _bin_body,
    out_type=(
        jax.ShapeDtypeStruct((NW, CAP), jnp.int32),
        jax.ShapeDtypeStruct((NW, CAP), jnp.int32),
        jax.ShapeDtypeStruct((NW, 16), jnp.int32),
    ),
    mesh=_mesh,
    compiler_params=pltpu.CompilerParams(needs_layout_passes=False),
    scratch_types=[
        pltpu.VMEM((CAP,), jnp.int32),
        pltpu.VMEM((CAP,), jnp.int32),
        pltpu.VMEM((EC,), jnp.int32),
        pltpu.VMEM((EC,), jnp.int32),
        pltpu.VMEM((16,), jnp.int32),
    ],
)


def _segmax_body(h_hbm, srcb_hbm, dstb_hbm, cnt_hbm, out_hbm, srcbin2,
                 dstbin, rowbuf, aggr, cntv, sem, *, fixinf):
    wid = _wid()
    lo = wid * RANGE
    iota = lax.iota(jnp.int32, 16)

    pltpu.sync_copy(cnt_hbm.at[wid], cntv)
    pltpu.sync_copy(srcb_hbm.at[wid], srcbin2)
    pltpu.sync_copy(dstb_hbm.at[wid], dstbin)
    cnt = jnp.max(cntv[...])

    neg = jnp.full((16,), -jnp.inf, jnp.float32)

    def initr(r, _):
        for j in range(8):
            aggr[r, pl.ds(j * 16, 16)] = neg
        return 0

    lax.fori_loop(0, RANGE + 1, initr, 0)

    def chunk(g, _):
        pltpu.async_copy(h_hbm.at[srcbin2.at[g]], rowbuf, sem).wait()

        def sub(s, _):
            dstv = dstbin[pl.ds(g * G + s * 16, 16)]
            for e in range(16):
                d = jnp.max(jnp.where(iota == e, dstv, 0))
                row = s * 16 + e
                for j in range(8):
                    sl = pl.ds(j * 16, 16)
                    aggr[d, sl] = jnp.maximum(aggr[d, sl], rowbuf[row, sl])
            return 0

        lax.fori_loop(0, G // 16, sub, 0)
        return 0

    lax.fori_loop(0, cnt // G, chunk, 0)

    if fixinf:
        def fixr(r, _):
            for j in range(8):
                sl = pl.ds(j * 16, 16)
                v = aggr[r, sl]
                aggr[r, sl] = jnp.where(v == -jnp.inf,
                                        jnp.zeros((16,), jnp.float32), v)
            return 0

        lax.fori_loop(0, RANGE, fixr, 0)

    pltpu.sync_copy(aggr.at[pl.ds(0, RANGE)], out_hbm.at[pl.ds(lo, RANGE)])


def _make_segmax(fixinf):
    return pl.kernel(
        functools.partial(_segmax_body, fixinf=fixinf),
        out_type=jax.ShapeDtypeStruct((NP, D), jnp.float32),
        mesh=_mesh,
        compiler_params=pltpu.CompilerParams(needs_layout_passes=False),
        scratch_types=[
            pltpu.VMEM((CAP // G, G), jnp.int32),
            pltpu.VMEM((CAP,), jnp.int32),
            pltpu.VMEM((G, D), jnp.float32),
            pltpu.VMEM((RANGE + 1, D), jnp.float32),
            pltpu.VMEM((16,), jnp.int32),
            pltpu.SemaphoreType.DMA,
        ],
    )


_segmax_raw = _make_segmax(False)
_segmax_fix = _make_segmax(True)

BM = 1280


def _mm_body(x_ref, w_ref, b_ref, o_ref, *, relu_in):
    xv = x_ref[...]
    if relu_in:
        xv = jnp.maximum(xv, 0.0)
    acc = lax.dot_general(xv, w_ref[...], (((1,), (1,)), ((), ())),
                          preferred_element_type=jnp.float32)
    o_ref[...] = acc + b_ref[0:1, :]


def _mm(x, w, b8, relu_in):
    return pl.pallas_call(
        functools.partial(_mm_body, relu_in=relu_in),
        grid=(NP // BM,),
        in_specs=[
            pl.BlockSpec((BM, D), lambda i: (i, 0)),
            pl.BlockSpec((D, D), lambda i: (0, 0)),
            pl.BlockSpec((8, D), lambda i: (0, 0)),
        ],
        out_specs=pl.BlockSpec((BM, D), lambda i: (i, 0)),
        out_shape=jax.ShapeDtypeStruct((NP, D), jnp.float32),
    )(x, w, b8)


@jax.jit
def kernel(x, edge_index, W1, b1, W2, b2):
    x_pad = jnp.pad(x, ((0, NP - N), (0, 0)))
    b1_8 = jnp.broadcast_to(b1[None, :], (8, D))
    b2_8 = jnp.broadcast_to(b2[None, :], (8, D))

    srcb, dstb, cnts = _bin_call(edge_index.reshape(-1))
    srcb3 = srcb.reshape(NW, CAP // G, G)

    h1 = _mm(x_pad, W1, b1_8, relu_in=False)
    agg1 = _segmax_raw(h1, srcb3, dstb, cnts)
    h2 = _mm(agg1, W2, b2_8, relu_in=True)
    out = _segmax_fix(h2, srcb3, dstb, cnts)
    return out[:N]
